# Initial kernel scaffold; baseline (speedup 1.0000x reference)
import jax
import jax.numpy as jnp
from jax.experimental import pallas as pl


def kernel(x, Wg, bg, W1, b1, W2, b2):
    raise NotImplementedError("write your pallas kernel here")


# speedup vs baseline: 2.4576x; 2.4576x over previous
import functools

import jax
import jax.numpy as jnp
from jax import lax
from jax.experimental import pallas as pl
from jax.experimental.pallas import tpu as pltpu
from jax.experimental.pallas import tpu_sc as plsc

N = 4096
D = 1024
E = 16
K = 2
H = 2048

TB = 256
NG = N // TB
B = 128
NB = N * K // B + E
PMAX = NB * B

NWORK = 32
TPW = N // NWORK
DCH = 64
CCH = 32


def _gating_body(x_ref, wg_ref, bg_ref,
                 probs_ref, mask_ref, i0_ref, i1_ref, r0_ref, r1_ref,
                 w0_ref, w1_ref, counts_ref, aux_ref,
                 run_ref, acc_ref):
    b = pl.program_id(0)

    @pl.when(b == 0)
    def _init():
        run_ref[...] = jnp.zeros_like(run_ref)
        acc_ref[...] = jnp.zeros_like(acc_ref)

    x = x_ref[...]
    logits = jnp.dot(x, wg_ref[...], preferred_element_type=jnp.float32)
    logits = logits + bg_ref[...]
    m = jnp.max(logits, axis=1, keepdims=True)
    p = jnp.exp(logits - m)
    probs = p / jnp.sum(p, axis=1, keepdims=True)
    probs_ref[...] = probs

    iota = lax.broadcasted_iota(jnp.int32, (TB, E), 1)
    m0 = jnp.max(probs, axis=1, keepdims=True)
    i0 = jnp.min(jnp.where(probs == m0, iota, E), axis=1, keepdims=True)
    oh0 = iota == i0
    probs_m = jnp.where(oh0, -1.0, probs)
    m1 = jnp.max(probs_m, axis=1, keepdims=True)
    i1 = jnp.min(jnp.where(probs_m == m1, iota, E), axis=1, keepdims=True)
    oh1 = iota == i1

    mask_ref[...] = (oh0 | oh1).astype(jnp.float32)
    s = m0 + m1 + 1e-9
    w0_ref[...] = m0 / s
    w1_ref[...] = m1 / s
    i0_ref[...] = i0
    i1_ref[...] = i1

    oh0f = oh0.astype(jnp.float32)
    oh1f = oh1.astype(jnp.float32)
    sel = oh0f + oh1f
    ri = lax.broadcasted_iota(jnp.int32, (TB, TB), 0)
    ci = lax.broadcasted_iota(jnp.int32, (TB, TB), 1)
    tril = (ci < ri).astype(jnp.float32)
    excl = jnp.dot(tril, sel, preferred_element_type=jnp.float32) + run_ref[...]
    r0_ref[...] = jnp.sum(excl * oh0f, axis=1, keepdims=True).astype(jnp.int32)
    r1_ref[...] = jnp.sum(excl * oh1f, axis=1, keepdims=True).astype(jnp.int32)
    run_ref[...] = run_ref[...] + jnp.sum(sel, axis=0, keepdims=True)
    acc_ref[...] = acc_ref[...] + jnp.sum(probs, axis=0, keepdims=True)
    counts_ref[...] = run_ref[...].astype(jnp.int32)

    @pl.when(b == NG - 1)
    def _fin():
        mean = acc_ref[...] / N
        u = 1.0 / E
        aux = jnp.sum(u * (jnp.log(u) - jnp.log(mean))) / E
        aux_ref[...] = jnp.full((1, 1), aux, jnp.float32)


def _gating(x, Wg, bg):
    out_shapes = (
        jax.ShapeDtypeStruct((N, E), jnp.float32),
        jax.ShapeDtypeStruct((N, E), jnp.float32),
        jax.ShapeDtypeStruct((N, 1), jnp.int32),
        jax.ShapeDtypeStruct((N, 1), jnp.int32),
        jax.ShapeDtypeStruct((N, 1), jnp.int32),
        jax.ShapeDtypeStruct((N, 1), jnp.int32),
        jax.ShapeDtypeStruct((N, 1), jnp.float32),
        jax.ShapeDtypeStruct((N, 1), jnp.float32),
        jax.ShapeDtypeStruct((1, E), jnp.int32),
        jax.ShapeDtypeStruct((1, 1), jnp.float32),
    )
    ne_spec = pl.BlockSpec((TB, E), lambda b: (b, 0))
    n1_spec = pl.BlockSpec((TB, 1), lambda b: (b, 0))
    one_spec = pl.BlockSpec((1, E), lambda b: (0, 0))
    return pl.pallas_call(
        _gating_body,
        grid=(NG,),
        in_specs=[
            pl.BlockSpec((TB, D), lambda b: (b, 0)),
            pl.BlockSpec((D, E), lambda b: (0, 0)),
            one_spec,
        ],
        out_specs=(ne_spec, ne_spec, n1_spec, n1_spec, n1_spec, n1_spec,
                   n1_spec, n1_spec, one_spec,
                   pl.BlockSpec((1, 1), lambda b: (0, 0))),
        out_shape=out_shapes,
        scratch_shapes=[
            pltpu.VMEM((1, E), jnp.float32),
            pltpu.VMEM((1, E), jnp.float32),
        ],
    )(x, Wg, bg.reshape(1, E))


def _dispatch_body(x_hbm, i0_hbm, i1_hbm, r0_hbm, r1_hbm, off_hbm,
                   xs_hbm, pos0_hbm, pos1_hbm,
                   xrows, e_v, r_v, p0_v, p1_v, off_v, sem):
    wid = lax.axis_index("s") * 2 + lax.axis_index("c")
    pltpu.sync_copy(off_hbm, off_v)

    def calc_pos(base, e_hbm, rk_hbm, pos_hbm, p_v):
        pltpu.sync_copy(e_hbm.at[pl.ds(base, DCH)], e_v)
        pltpu.sync_copy(rk_hbm.at[pl.ds(base, DCH)], r_v)
        for i in range(DCH // 16):
            sl = pl.ds(i * 16, 16)
            off = plsc.load_gather(off_v, [e_v[sl]])
            p_v[sl] = off + r_v[sl]
        pltpu.sync_copy(p_v, pos_hbm.at[pl.ds(base, DCH)])

    for c in range(TPW // DCH):
        base = wid * TPW + c * DCH
        pltpu.sync_copy(x_hbm.at[pl.ds(base, DCH)], xrows)
        calc_pos(base, i0_hbm, r0_hbm, pos0_hbm, p0_v)
        calc_pos(base, i1_hbm, r1_hbm, pos1_hbm, p1_v)
        d0 = pltpu.async_copy(xrows, xs_hbm.at[p0_v], sem)
        d1 = pltpu.async_copy(xrows, xs_hbm.at[p1_v], sem)
        d0.wait()
        d1.wait()


def _dispatch(x, i0, i1, r0, r1, off):
    mesh = plsc.VectorSubcoreMesh(core_axis_name="c", subcore_axis_name="s")
    f = functools.partial(
        pl.kernel, _dispatch_body, mesh=mesh,
        out_type=(jax.ShapeDtypeStruct((PMAX, D), jnp.float32),
                  jax.ShapeDtypeStruct((N,), jnp.int32),
                  jax.ShapeDtypeStruct((N,), jnp.int32)),
        scratch_types=[
            pltpu.VMEM((DCH, D), jnp.float32),
            pltpu.VMEM((DCH,), jnp.int32),
            pltpu.VMEM((DCH,), jnp.int32),
            pltpu.VMEM((DCH,), jnp.int32),
            pltpu.VMEM((DCH,), jnp.int32),
            pltpu.VMEM((E,), jnp.int32),
            pltpu.SemaphoreType.DMA,
        ],
        compiler_params=pltpu.CompilerParams(needs_layout_passes=False),
    )()
    return f(x, i0, i1, r0, r1, off)


NSLOT = 3
EV = 32


def _mlp_body(be_r, tr_r, sl_r, pfe_r, pfv_r, pfs_r, ev_r,
              xs_ref, b1_ref, b2_ref, w1_any, w2_any, ys_ref,
              w1b, w2b, sems):
    b = pl.program_id(0)
    nv = ev_r[EV]

    def fetch(e, s):
        pltpu.make_async_copy(w1_any.at[e], w1b.at[s], sems.at[s]).start()
        pltpu.make_async_copy(w2_any.at[e], w2b.at[s], sems.at[s]).start()

    @pl.when(b == 0)
    def _prime():
        fetch(ev_r[0], 0)

        @pl.when(nv >= 2)
        def _p1():
            fetch(ev_r[1], 1)

        @pl.when(nv >= 3)
        def _p2():
            fetch(ev_r[2], 2)

    sl = sl_r[b]

    @pl.when(tr_r[b] == 1)
    def _on_transition():
        @pl.when((b > 0) & (pfv_r[b] == 1))
        def _pf():
            fetch(pfe_r[b], pfs_r[b])

        pltpu.make_async_copy(w1_any.at[be_r[b]], w1b.at[sl], sems.at[sl]).wait()
        pltpu.make_async_copy(w2_any.at[be_r[b]], w2b.at[sl], sems.at[sl]).wait()

    x = xs_ref[...]
    h = jnp.dot(x, w1b[sl], preferred_element_type=jnp.float32)
    h = jnp.maximum(h + b1_ref[0], 0.0)
    ys_ref[...] = jnp.dot(h, w2b[sl], preferred_element_type=jnp.float32) + b2_ref[0]


def _mlp(block_expert, xs, W1, b1, W2, b2):
    i32 = jnp.int32
    trans = jnp.concatenate([
        jnp.ones((1,), i32),
        (block_expert[1:] != block_expert[:-1]).astype(i32)])
    visit = jnp.cumsum(trans) - 1
    slot = (visit % NSLOT).astype(i32)
    nv = visit[-1] + 1
    ev = jnp.zeros((EV,), i32).at[jnp.minimum(visit, EV - 1)].set(block_expert)
    pf_v = visit + 2
    pf_valid = (pf_v < nv).astype(i32)
    pf_e = ev[jnp.minimum(pf_v, EV - 1)]
    pf_slot = (pf_v % NSLOT).astype(i32)
    evnv = jnp.concatenate([ev, nv.reshape(1)]).astype(i32)

    grid_spec = pltpu.PrefetchScalarGridSpec(
        num_scalar_prefetch=7,
        grid=(NB,),
        in_specs=[
            pl.BlockSpec((B, D), lambda b, *s: (b, 0)),
            pl.BlockSpec((1, 1, H), lambda b, be, *s: (be[b], 0, 0)),
            pl.BlockSpec((1, 1, D), lambda b, be, *s: (be[b], 0, 0)),
            pl.BlockSpec(memory_space=pl.ANY),
            pl.BlockSpec(memory_space=pl.ANY),
        ],
        out_specs=pl.BlockSpec((B, D), lambda b, *s: (b, 0)),
        scratch_shapes=[
            pltpu.VMEM((NSLOT, D, H), jnp.float32),
            pltpu.VMEM((NSLOT, H, D), jnp.float32),
            pltpu.SemaphoreType.DMA((NSLOT,)),
        ],
    )
    return pl.pallas_call(
        _mlp_body,
        grid_spec=grid_spec,
        out_shape=jax.ShapeDtypeStruct((PMAX, D), jnp.float32),
        compiler_params=pltpu.CompilerParams(vmem_limit_bytes=110 * 1024 * 1024),
    )(block_expert, trans, slot, pf_e, pf_valid, pf_slot, evnv,
      xs, b1.reshape(E, 1, H), b2.reshape(E, 1, D), W1, W2)


def _combine_body(ys_hbm, pos0_hbm, pos1_hbm, w0_hbm, w1_hbm, y_hbm,
                  buf0, buf1, outb, p0_v, p1_v, w0_v, w1_v, sem):
    wid = lax.axis_index("s") * 2 + lax.axis_index("c")

    for c in range(TPW // CCH):
        base = wid * TPW + c * CCH
        pltpu.sync_copy(pos0_hbm.at[pl.ds(base, CCH)], p0_v)
        pltpu.sync_copy(pos1_hbm.at[pl.ds(base, CCH)], p1_v)
        pltpu.sync_copy(w0_hbm.at[pl.ds(base, CCH)], w0_v)
        pltpu.sync_copy(w1_hbm.at[pl.ds(base, CCH)], w1_v)
        g0 = pltpu.async_copy(ys_hbm.at[p0_v], buf0, sem)
        g1 = pltpu.async_copy(ys_hbm.at[p1_v], buf1, sem)
        g0.wait()
        g1.wait()

        def tbody(t, carry):
            tv = jnp.full((16,), t, jnp.int32)
            wb0 = plsc.load_gather(w0_v, [tv])
            wb1 = plsc.load_gather(w1_v, [tv])
            for cc in range(D // 16):
                sl = pl.ds(cc * 16, 16)
                outb[t, sl] = wb0 * buf0[t, sl] + wb1 * buf1[t, sl]
            return carry

        lax.fori_loop(0, CCH, tbody, 0)
        pltpu.sync_copy(outb, y_hbm.at[pl.ds(base, CCH)])


def _combine(ys, pos0, pos1, w0, w1):
    mesh = plsc.VectorSubcoreMesh(core_axis_name="c", subcore_axis_name="s")
    f = functools.partial(
        pl.kernel, _combine_body, mesh=mesh,
        out_type=jax.ShapeDtypeStruct((N, D), jnp.float32),
        scratch_types=[
            pltpu.VMEM((CCH, D), jnp.float32),
            pltpu.VMEM((CCH, D), jnp.float32),
            pltpu.VMEM((CCH, D), jnp.float32),
            pltpu.VMEM((CCH,), jnp.int32),
            pltpu.VMEM((CCH,), jnp.int32),
            pltpu.VMEM((CCH,), jnp.float32),
            pltpu.VMEM((CCH,), jnp.float32),
            pltpu.SemaphoreType.DMA,
        ],
        compiler_params=pltpu.CompilerParams(needs_layout_passes=False),
    )()
    return f(ys, pos0, pos1, w0, w1)


def kernel(x, Wg, bg, W1, b1, W2, b2):
    (probs, mask, i0, i1, r0, r1, w0, w1, counts, aux) = _gating(x, Wg, bg)

    counts = counts[0]
    padded = ((counts + B - 1) // B) * B
    csum = jnp.cumsum(padded)
    off = (csum - padded).astype(jnp.int32)
    blk_end = (csum // B).astype(jnp.int32)
    bids = jnp.arange(NB, dtype=jnp.int32)
    block_expert = jnp.minimum(
        jnp.sum((bids[:, None] >= blk_end[None, :]).astype(jnp.int32), axis=1),
        E - 1).astype(jnp.int32)

    i0f = i0.reshape(N)
    i1f = i1.reshape(N)
    r0f = r0.reshape(N)
    r1f = r1.reshape(N)

    xs, pos0, pos1 = _dispatch(x, i0f, i1f, r0f, r1f, off)
    ys = _mlp(block_expert, xs, W1, b1, W2, b2)
    y = _combine(ys, pos0, pos1, w0.reshape(N), w1.reshape(N))

    return (y, aux.reshape(()), probs, mask)


# speedup vs baseline: 2.5129x; 1.0225x over previous
import functools

import jax
import jax.numpy as jnp
from jax import lax
from jax.experimental import pallas as pl
from jax.experimental.pallas import tpu as pltpu
from jax.experimental.pallas import tpu_sc as plsc

N = 4096
D = 1024
E = 16
K = 2
H = 2048

TB = 256
NG = N // TB
B = 128
NB = N * K // B + E
PMAX = NB * B

NWORK = 32
TPW = N // NWORK
DCH = 64
CCH = 16


def _gating_body(x_ref, wg_ref, bg_ref,
                 probs_ref, mask_ref, i0_ref, i1_ref, r0_ref, r1_ref,
                 w0_ref, w1_ref, counts_ref, aux_ref,
                 run_ref, acc_ref):
    b = pl.program_id(0)

    @pl.when(b == 0)
    def _init():
        run_ref[...] = jnp.zeros_like(run_ref)
        acc_ref[...] = jnp.zeros_like(acc_ref)

    x = x_ref[...]
    logits = jnp.dot(x, wg_ref[...], preferred_element_type=jnp.float32)
    logits = logits + bg_ref[...]
    m = jnp.max(logits, axis=1, keepdims=True)
    p = jnp.exp(logits - m)
    probs = p / jnp.sum(p, axis=1, keepdims=True)
    probs_ref[...] = probs

    iota = lax.broadcasted_iota(jnp.int32, (TB, E), 1)
    m0 = jnp.max(probs, axis=1, keepdims=True)
    i0 = jnp.min(jnp.where(probs == m0, iota, E), axis=1, keepdims=True)
    oh0 = iota == i0
    probs_m = jnp.where(oh0, -1.0, probs)
    m1 = jnp.max(probs_m, axis=1, keepdims=True)
    i1 = jnp.min(jnp.where(probs_m == m1, iota, E), axis=1, keepdims=True)
    oh1 = iota == i1

    mask_ref[...] = (oh0 | oh1).astype(jnp.float32)
    s = m0 + m1 + 1e-9
    w0_ref[...] = (m0 / s) + jnp.zeros((TB, E), jnp.float32)
    w1_ref[...] = (m1 / s) + jnp.zeros((TB, E), jnp.float32)
    i0_ref[...] = i0
    i1_ref[...] = i1

    oh0f = oh0.astype(jnp.float32)
    oh1f = oh1.astype(jnp.float32)
    sel = oh0f + oh1f
    ri = lax.broadcasted_iota(jnp.int32, (TB, TB), 0)
    ci = lax.broadcasted_iota(jnp.int32, (TB, TB), 1)
    tril = (ci < ri).astype(jnp.float32)
    excl = jnp.dot(tril, sel, preferred_element_type=jnp.float32) + run_ref[...]
    r0_ref[...] = jnp.sum(excl * oh0f, axis=1, keepdims=True).astype(jnp.int32)
    r1_ref[...] = jnp.sum(excl * oh1f, axis=1, keepdims=True).astype(jnp.int32)
    run_ref[...] = run_ref[...] + jnp.sum(sel, axis=0, keepdims=True)
    acc_ref[...] = acc_ref[...] + jnp.sum(probs, axis=0, keepdims=True)
    counts_ref[...] = run_ref[...].astype(jnp.int32)

    @pl.when(b == NG - 1)
    def _fin():
        mean = acc_ref[...] / N
        u = 1.0 / E
        aux = jnp.sum(u * (jnp.log(u) - jnp.log(mean))) / E
        aux_ref[...] = jnp.full((1, 1), aux, jnp.float32)


def _gating(x, Wg, bg):
    out_shapes = (
        jax.ShapeDtypeStruct((N, E), jnp.float32),
        jax.ShapeDtypeStruct((N, E), jnp.float32),
        jax.ShapeDtypeStruct((N, 1), jnp.int32),
        jax.ShapeDtypeStruct((N, 1), jnp.int32),
        jax.ShapeDtypeStruct((N, 1), jnp.int32),
        jax.ShapeDtypeStruct((N, 1), jnp.int32),
        jax.ShapeDtypeStruct((N, E), jnp.float32),
        jax.ShapeDtypeStruct((N, E), jnp.float32),
        jax.ShapeDtypeStruct((1, E), jnp.int32),
        jax.ShapeDtypeStruct((1, 1), jnp.float32),
    )
    ne_spec = pl.BlockSpec((TB, E), lambda b: (b, 0))
    n1_spec = pl.BlockSpec((TB, 1), lambda b: (b, 0))
    one_spec = pl.BlockSpec((1, E), lambda b: (0, 0))
    return pl.pallas_call(
        _gating_body,
        grid=(NG,),
        in_specs=[
            pl.BlockSpec((TB, D), lambda b: (b, 0)),
            pl.BlockSpec((D, E), lambda b: (0, 0)),
            one_spec,
        ],
        out_specs=(ne_spec, ne_spec, n1_spec, n1_spec, n1_spec, n1_spec,
                   ne_spec, ne_spec, one_spec,
                   pl.BlockSpec((1, 1), lambda b: (0, 0))),
        out_shape=out_shapes,
        scratch_shapes=[
            pltpu.VMEM((1, E), jnp.float32),
            pltpu.VMEM((1, E), jnp.float32),
        ],
    )(x, Wg, bg.reshape(1, E))


def _dispatch_body(x_hbm, i0_hbm, i1_hbm, r0_hbm, r1_hbm, off_hbm,
                   xs_hbm, pos0_hbm, pos1_hbm,
                   xrows, e_v, r_v, p0_v, p1_v, off_v, sem):
    wid = lax.axis_index("s") * 2 + lax.axis_index("c")
    pltpu.sync_copy(off_hbm, off_v)

    def calc_pos(base, e_hbm, rk_hbm, pos_hbm, p_v):
        pltpu.sync_copy(e_hbm.at[pl.ds(base, DCH)], e_v)
        pltpu.sync_copy(rk_hbm.at[pl.ds(base, DCH)], r_v)
        for i in range(DCH // 16):
            sl = pl.ds(i * 16, 16)
            off = plsc.load_gather(off_v, [e_v[sl]])
            p_v[sl] = off + r_v[sl]
        pltpu.sync_copy(p_v, pos_hbm.at[pl.ds(base, DCH)])

    for c in range(TPW // DCH):
        base = wid * TPW + c * DCH
        pltpu.sync_copy(x_hbm.at[pl.ds(base, DCH)], xrows)
        calc_pos(base, i0_hbm, r0_hbm, pos0_hbm, p0_v)
        calc_pos(base, i1_hbm, r1_hbm, pos1_hbm, p1_v)
        d0 = pltpu.async_copy(xrows, xs_hbm.at[p0_v], sem)
        d1 = pltpu.async_copy(xrows, xs_hbm.at[p1_v], sem)
        d0.wait()
        d1.wait()


def _dispatch(x, i0, i1, r0, r1, off):
    mesh = plsc.VectorSubcoreMesh(core_axis_name="c", subcore_axis_name="s")
    f = functools.partial(
        pl.kernel, _dispatch_body, mesh=mesh,
        out_type=(jax.ShapeDtypeStruct((PMAX, D), jnp.float32),
                  jax.ShapeDtypeStruct((N,), jnp.int32),
                  jax.ShapeDtypeStruct((N,), jnp.int32)),
        scratch_types=[
            pltpu.VMEM((DCH, D), jnp.float32),
            pltpu.VMEM((DCH,), jnp.int32),
            pltpu.VMEM((DCH,), jnp.int32),
            pltpu.VMEM((DCH,), jnp.int32),
            pltpu.VMEM((DCH,), jnp.int32),
            pltpu.VMEM((E,), jnp.int32),
            pltpu.SemaphoreType.DMA,
        ],
        compiler_params=pltpu.CompilerParams(needs_layout_passes=False),
    )()
    return f(x, i0, i1, r0, r1, off)


NSLOT = 3
EV = 32


def _mlp_body(be_r, tr_r, sl_r, pfe_r, pfv_r, pfs_r, ev_r,
              xs_ref, b1_ref, b2_ref, w1_any, w2_any, ys_ref,
              w1b, w2b, sems):
    b = pl.program_id(0)
    nv = ev_r[EV]

    def fetch(e, s):
        pltpu.make_async_copy(w1_any.at[e], w1b.at[s], sems.at[s]).start()
        pltpu.make_async_copy(w2_any.at[e], w2b.at[s], sems.at[s]).start()

    @pl.when(b == 0)
    def _prime():
        fetch(ev_r[0], 0)

        @pl.when(nv >= 2)
        def _p1():
            fetch(ev_r[1], 1)

        @pl.when(nv >= 3)
        def _p2():
            fetch(ev_r[2], 2)

    sl = sl_r[b]

    @pl.when(tr_r[b] == 1)
    def _on_transition():
        @pl.when((b > 0) & (pfv_r[b] == 1))
        def _pf():
            fetch(pfe_r[b], pfs_r[b])

        pltpu.make_async_copy(w1_any.at[be_r[b]], w1b.at[sl], sems.at[sl]).wait()
        pltpu.make_async_copy(w2_any.at[be_r[b]], w2b.at[sl], sems.at[sl]).wait()

    x = xs_ref[...]
    h = jnp.dot(x, w1b[sl], preferred_element_type=jnp.float32)
    h = jnp.maximum(h + b1_ref[0], 0.0)
    ys_ref[...] = jnp.dot(h, w2b[sl], preferred_element_type=jnp.float32) + b2_ref[0]


def _mlp(block_expert, xs, W1, b1, W2, b2):
    i32 = jnp.int32
    trans = jnp.concatenate([
        jnp.ones((1,), i32),
        (block_expert[1:] != block_expert[:-1]).astype(i32)])
    visit = jnp.cumsum(trans) - 1
    slot = (visit % NSLOT).astype(i32)
    nv = visit[-1] + 1
    ev = jnp.zeros((EV,), i32).at[jnp.minimum(visit, EV - 1)].set(block_expert)
    pf_v = visit + 2
    pf_valid = (pf_v < nv).astype(i32)
    pf_e = ev[jnp.minimum(pf_v, EV - 1)]
    pf_slot = (pf_v % NSLOT).astype(i32)
    evnv = jnp.concatenate([ev, nv.reshape(1)]).astype(i32)

    grid_spec = pltpu.PrefetchScalarGridSpec(
        num_scalar_prefetch=7,
        grid=(NB,),
        in_specs=[
            pl.BlockSpec((B, D), lambda b, *s: (b, 0)),
            pl.BlockSpec((1, 1, H), lambda b, be, *s: (be[b], 0, 0)),
            pl.BlockSpec((1, 1, D), lambda b, be, *s: (be[b], 0, 0)),
            pl.BlockSpec(memory_space=pl.ANY),
            pl.BlockSpec(memory_space=pl.ANY),
        ],
        out_specs=pl.BlockSpec((B, D), lambda b, *s: (b, 0)),
        scratch_shapes=[
            pltpu.VMEM((NSLOT, D, H), jnp.float32),
            pltpu.VMEM((NSLOT, H, D), jnp.float32),
            pltpu.SemaphoreType.DMA((NSLOT,)),
        ],
    )
    return pl.pallas_call(
        _mlp_body,
        grid_spec=grid_spec,
        out_shape=jax.ShapeDtypeStruct((PMAX, D), jnp.float32),
        compiler_params=pltpu.CompilerParams(vmem_limit_bytes=110 * 1024 * 1024),
    )(block_expert, trans, slot, pf_e, pf_valid, pf_slot, evnv,
      xs, b1.reshape(E, 1, H), b2.reshape(E, 1, D), W1, W2)


NCH = TPW // CCH


def _combine_body(ys_hbm, pos0_hbm, pos1_hbm, w0_hbm, w1_hbm, y_hbm,
                  buf0, buf1, outb, p0a, p1a, w0m, w1m, gsem, osem):
    wid = lax.axis_index("s") * 2 + lax.axis_index("c")
    base = wid * TPW
    pltpu.sync_copy(pos0_hbm.at[pl.ds(base, TPW)], p0a)
    pltpu.sync_copy(pos1_hbm.at[pl.ds(base, TPW)], p1a)
    pltpu.sync_copy(w0_hbm.at[pl.ds(base * E, TPW * E)], w0m)
    pltpu.sync_copy(w1_hbm.at[pl.ds(base * E, TPW * E)], w1m)

    def fire(c):
        s = c % 2
        g0 = pltpu.async_copy(ys_hbm.at[p0a.at[pl.ds(c * CCH, CCH)]],
                              buf0.at[s], gsem.at[s])
        g1 = pltpu.async_copy(ys_hbm.at[p1a.at[pl.ds(c * CCH, CCH)]],
                              buf1.at[s], gsem.at[s])
        return (g0, g1)

    pend = {0: fire(0)}
    owr = {}
    for c in range(NCH):
        s = c % 2
        if c + 1 < NCH:
            pend[c + 1] = fire(c + 1)
        g0, g1 = pend.pop(c)
        g0.wait()
        g1.wait()
        if c - 2 in owr:
            owr.pop(c - 2).wait()

        def tbody(t, carry, c=c, s=s):
            wb0 = w0m[pl.ds((c * CCH + t) * E, 16)]
            wb1 = w1m[pl.ds((c * CCH + t) * E, 16)]
            for cc in range(D // 16):
                sl = pl.ds(cc * 16, 16)
                outb[s, t, sl] = wb0 * buf0[s, t, sl] + wb1 * buf1[s, t, sl]
            return carry

        lax.fori_loop(0, CCH, tbody, 0)
        owr[c] = pltpu.async_copy(
            outb.at[s], y_hbm.at[pl.ds(base + c * CCH, CCH)], osem.at[s])
    for c in sorted(owr):
        owr.pop(c).wait()


def _combine(ys, pos0, pos1, w0, w1):
    mesh = plsc.VectorSubcoreMesh(core_axis_name="c", subcore_axis_name="s")
    f = functools.partial(
        pl.kernel, _combine_body, mesh=mesh,
        out_type=jax.ShapeDtypeStruct((N, D), jnp.float32),
        scratch_types=[
            pltpu.VMEM((2, CCH, D), jnp.float32),
            pltpu.VMEM((2, CCH, D), jnp.float32),
            pltpu.VMEM((2, CCH, D), jnp.float32),
            pltpu.VMEM((TPW,), jnp.int32),
            pltpu.VMEM((TPW,), jnp.int32),
            pltpu.VMEM((TPW * E,), jnp.float32),
            pltpu.VMEM((TPW * E,), jnp.float32),
            pltpu.SemaphoreType.DMA((2,)),
            pltpu.SemaphoreType.DMA((2,)),
        ],
        compiler_params=pltpu.CompilerParams(needs_layout_passes=False),
    )()
    return f(ys, pos0, pos1, w0, w1)


def kernel(x, Wg, bg, W1, b1, W2, b2):
    (probs, mask, i0, i1, r0, r1, w0, w1, counts, aux) = _gating(x, Wg, bg)

    counts = counts[0]
    padded = ((counts + B - 1) // B) * B
    csum = jnp.cumsum(padded)
    off = (csum - padded).astype(jnp.int32)
    blk_end = (csum // B).astype(jnp.int32)
    bids = jnp.arange(NB, dtype=jnp.int32)
    block_expert = jnp.minimum(
        jnp.sum((bids[:, None] >= blk_end[None, :]).astype(jnp.int32), axis=1),
        E - 1).astype(jnp.int32)

    i0f = i0.reshape(N)
    i1f = i1.reshape(N)
    r0f = r0.reshape(N)
    r1f = r1.reshape(N)

    xs, pos0, pos1 = _dispatch(x, i0f, i1f, r0f, r1f, off)
    ys = _mlp(block_expert, xs, W1, b1, W2, b2)
    y = _combine(ys, pos0, pos1, w0.reshape(N * E), w1.reshape(N * E))

    return (y, aux.reshape(()), probs, mask)


# speedup vs baseline: 2.5747x; 1.0246x over previous
import functools

import jax
import jax.numpy as jnp
from jax import lax
from jax.experimental import pallas as pl
from jax.experimental.pallas import tpu as pltpu
from jax.experimental.pallas import tpu_sc as plsc

N = 4096
D = 1024
E = 16
K = 2
H = 2048

TB = 256
NG = N // TB
B = 128
NB = N * K // B + E
PMAX = NB * B

NWORK = 32
TPW = N // NWORK
DCH = 32
CCH = 16


def _gating_body(x_ref, wg_ref, bg_ref,
                 probs_ref, mask_ref, i0_ref, i1_ref, r0_ref, r1_ref,
                 w0_ref, w1_ref, counts_ref, aux_ref,
                 run_ref, acc_ref):
    b = pl.program_id(0)

    @pl.when(b == 0)
    def _init():
        run_ref[...] = jnp.zeros_like(run_ref)
        acc_ref[...] = jnp.zeros_like(acc_ref)

    x = x_ref[...]
    logits = jnp.dot(x, wg_ref[...], preferred_element_type=jnp.float32)
    logits = logits + bg_ref[...]
    m = jnp.max(logits, axis=1, keepdims=True)
    p = jnp.exp(logits - m)
    probs = p / jnp.sum(p, axis=1, keepdims=True)
    probs_ref[...] = probs

    iota = lax.broadcasted_iota(jnp.int32, (TB, E), 1)
    m0 = jnp.max(probs, axis=1, keepdims=True)
    i0 = jnp.min(jnp.where(probs == m0, iota, E), axis=1, keepdims=True)
    oh0 = iota == i0
    probs_m = jnp.where(oh0, -1.0, probs)
    m1 = jnp.max(probs_m, axis=1, keepdims=True)
    i1 = jnp.min(jnp.where(probs_m == m1, iota, E), axis=1, keepdims=True)
    oh1 = iota == i1

    mask_ref[...] = (oh0 | oh1).astype(jnp.float32)
    s = m0 + m1 + 1e-9
    w0_ref[...] = (m0 / s) + jnp.zeros((TB, E), jnp.float32)
    w1_ref[...] = (m1 / s) + jnp.zeros((TB, E), jnp.float32)
    i0_ref[...] = i0
    i1_ref[...] = i1

    oh0f = oh0.astype(jnp.float32)
    oh1f = oh1.astype(jnp.float32)
    sel = oh0f + oh1f
    ri = lax.broadcasted_iota(jnp.int32, (TB, TB), 0)
    ci = lax.broadcasted_iota(jnp.int32, (TB, TB), 1)
    tril = (ci < ri).astype(jnp.float32)
    excl = jnp.dot(tril, sel, preferred_element_type=jnp.float32) + run_ref[...]
    r0_ref[...] = jnp.sum(excl * oh0f, axis=1, keepdims=True).astype(jnp.int32)
    r1_ref[...] = jnp.sum(excl * oh1f, axis=1, keepdims=True).astype(jnp.int32)
    run_ref[...] = run_ref[...] + jnp.sum(sel, axis=0, keepdims=True)
    acc_ref[...] = acc_ref[...] + jnp.sum(probs, axis=0, keepdims=True)
    counts_ref[...] = run_ref[...].astype(jnp.int32)

    @pl.when(b == NG - 1)
    def _fin():
        mean = acc_ref[...] / N
        u = 1.0 / E
        aux = jnp.sum(u * (jnp.log(u) - jnp.log(mean))) / E
        aux_ref[...] = jnp.full((1, 1), aux, jnp.float32)


def _gating(x, Wg, bg):
    out_shapes = (
        jax.ShapeDtypeStruct((N, E), jnp.float32),
        jax.ShapeDtypeStruct((N, E), jnp.float32),
        jax.ShapeDtypeStruct((N, 1), jnp.int32),
        jax.ShapeDtypeStruct((N, 1), jnp.int32),
        jax.ShapeDtypeStruct((N, 1), jnp.int32),
        jax.ShapeDtypeStruct((N, 1), jnp.int32),
        jax.ShapeDtypeStruct((N, E), jnp.float32),
        jax.ShapeDtypeStruct((N, E), jnp.float32),
        jax.ShapeDtypeStruct((1, E), jnp.int32),
        jax.ShapeDtypeStruct((1, 1), jnp.float32),
    )
    ne_spec = pl.BlockSpec((TB, E), lambda b: (b, 0))
    n1_spec = pl.BlockSpec((TB, 1), lambda b: (b, 0))
    one_spec = pl.BlockSpec((1, E), lambda b: (0, 0))
    return pl.pallas_call(
        _gating_body,
        grid=(NG,),
        in_specs=[
            pl.BlockSpec((TB, D), lambda b: (b, 0)),
            pl.BlockSpec((D, E), lambda b: (0, 0)),
            one_spec,
        ],
        out_specs=(ne_spec, ne_spec, n1_spec, n1_spec, n1_spec, n1_spec,
                   ne_spec, ne_spec, one_spec,
                   pl.BlockSpec((1, 1), lambda b: (0, 0))),
        out_shape=out_shapes,
        scratch_shapes=[
            pltpu.VMEM((1, E), jnp.float32),
            pltpu.VMEM((1, E), jnp.float32),
        ],
    )(x, Wg, bg.reshape(1, E))


def _dispatch_body(x_hbm, i0_hbm, i1_hbm, r0_hbm, r1_hbm, off_hbm,
                   xs_hbm, pos0_hbm, pos1_hbm,
                   xrows, e_v, r_v, p0b, p1b, off_v, sem):
    wid = lax.axis_index("s") * 2 + lax.axis_index("c")
    pltpu.sync_copy(off_hbm, off_v)

    def calc_pos(base, s, e_hbm, rk_hbm, pos_hbm, p_b):
        pltpu.sync_copy(e_hbm.at[pl.ds(base, DCH)], e_v)
        pltpu.sync_copy(rk_hbm.at[pl.ds(base, DCH)], r_v)
        for i in range(DCH // 16):
            sl = pl.ds(i * 16, 16)
            off = plsc.load_gather(off_v, [e_v[sl]])
            p_b[s, sl] = off + r_v[sl]
        pltpu.sync_copy(p_b.at[s], pos_hbm.at[pl.ds(base, DCH)])

    pend = {}
    for c in range(TPW // DCH):
        s = c % 2
        base = wid * TPW + c * DCH
        if c - 2 in pend:
            d0, d1 = pend.pop(c - 2)
            d0.wait()
            d1.wait()
        pltpu.sync_copy(x_hbm.at[pl.ds(base, DCH)], xrows.at[s])
        calc_pos(base, s, i0_hbm, r0_hbm, pos0_hbm, p0b)
        calc_pos(base, s, i1_hbm, r1_hbm, pos1_hbm, p1b)
        pend[c] = (pltpu.async_copy(xrows.at[s], xs_hbm.at[p0b.at[s]], sem.at[s]),
                   pltpu.async_copy(xrows.at[s], xs_hbm.at[p1b.at[s]], sem.at[s]))
    for c in sorted(pend):
        d0, d1 = pend.pop(c)
        d0.wait()
        d1.wait()


def _dispatch(x, i0, i1, r0, r1, off):
    mesh = plsc.VectorSubcoreMesh(core_axis_name="c", subcore_axis_name="s")
    f = functools.partial(
        pl.kernel, _dispatch_body, mesh=mesh,
        out_type=(jax.ShapeDtypeStruct((PMAX, D), jnp.float32),
                  jax.ShapeDtypeStruct((N,), jnp.int32),
                  jax.ShapeDtypeStruct((N,), jnp.int32)),
        scratch_types=[
            pltpu.VMEM((2, DCH, D), jnp.float32),
            pltpu.VMEM((DCH,), jnp.int32),
            pltpu.VMEM((DCH,), jnp.int32),
            pltpu.VMEM((2, DCH), jnp.int32),
            pltpu.VMEM((2, DCH), jnp.int32),
            pltpu.VMEM((E,), jnp.int32),
            pltpu.SemaphoreType.DMA((2,)),
        ],
        compiler_params=pltpu.CompilerParams(needs_layout_passes=False),
    )()
    return f(x, i0, i1, r0, r1, off)


NSLOT = 3
EV = 32


def _mlp_body(be_r, tr_r, sl_r, pfe_r, pfv_r, pfs_r, ev_r,
              xs_ref, b1_ref, b2_ref, w1_any, w2_any, ys_ref,
              w1b, w2b, sems):
    b = pl.program_id(0)
    nv = ev_r[EV]
    used = ev_r[EV + 1]

    def fetch(e, s):
        pltpu.make_async_copy(w1_any.at[e], w1b.at[s], sems.at[s]).start()
        pltpu.make_async_copy(w2_any.at[e], w2b.at[s], sems.at[s]).start()

    @pl.when(b == 0)
    def _prime():
        fetch(ev_r[0], 0)

        @pl.when(nv >= 2)
        def _p1():
            fetch(ev_r[1], 1)

        @pl.when(nv >= 3)
        def _p2():
            fetch(ev_r[2], 2)

    sl = sl_r[b]

    @pl.when(tr_r[b] == 1)
    def _on_transition():
        @pl.when((b > 0) & (pfv_r[b] == 1))
        def _pf():
            fetch(pfe_r[b], pfs_r[b])

        pltpu.make_async_copy(w1_any.at[be_r[b]], w1b.at[sl], sems.at[sl]).wait()
        pltpu.make_async_copy(w2_any.at[be_r[b]], w2b.at[sl], sems.at[sl]).wait()

    @pl.when(b < used)
    def _compute():
        x = xs_ref[...]
        h = jnp.dot(x, w1b[sl], preferred_element_type=jnp.float32)
        h = jnp.maximum(h + b1_ref[0], 0.0)
        ys_ref[...] = (jnp.dot(h, w2b[sl], preferred_element_type=jnp.float32)
                       + b2_ref[0])


def _mlp(block_expert, used, xs, W1, b1, W2, b2):
    i32 = jnp.int32
    trans = jnp.concatenate([
        jnp.ones((1,), i32),
        (block_expert[1:] != block_expert[:-1]).astype(i32)])
    visit = jnp.cumsum(trans) - 1
    slot = (visit % NSLOT).astype(i32)
    nv = visit[-1] + 1
    ev = jnp.zeros((EV,), i32).at[jnp.minimum(visit, EV - 1)].set(block_expert)
    pf_v = visit + 2
    pf_valid = (pf_v < nv).astype(i32)
    pf_e = ev[jnp.minimum(pf_v, EV - 1)]
    pf_slot = (pf_v % NSLOT).astype(i32)
    evnv = jnp.concatenate([ev, nv.reshape(1), used.reshape(1)]).astype(i32)

    def data_idx(b, *s):
        ev_r = s[6]
        return (jnp.minimum(b, ev_r[EV + 1] - 1), 0)

    grid_spec = pltpu.PrefetchScalarGridSpec(
        num_scalar_prefetch=7,
        grid=(NB,),
        in_specs=[
            pl.BlockSpec((B, D), data_idx),
            pl.BlockSpec((1, 1, H), lambda b, be, *s: (be[b], 0, 0)),
            pl.BlockSpec((1, 1, D), lambda b, be, *s: (be[b], 0, 0)),
            pl.BlockSpec(memory_space=pl.ANY),
            pl.BlockSpec(memory_space=pl.ANY),
        ],
        out_specs=pl.BlockSpec((B, D), data_idx),
        scratch_shapes=[
            pltpu.VMEM((NSLOT, D, H), jnp.float32),
            pltpu.VMEM((NSLOT, H, D), jnp.float32),
            pltpu.SemaphoreType.DMA((NSLOT,)),
        ],
    )
    return pl.pallas_call(
        _mlp_body,
        grid_spec=grid_spec,
        out_shape=jax.ShapeDtypeStruct((PMAX, D), jnp.float32),
        compiler_params=pltpu.CompilerParams(vmem_limit_bytes=110 * 1024 * 1024),
    )(block_expert, trans, slot, pf_e, pf_valid, pf_slot, evnv,
      xs, b1.reshape(E, 1, H), b2.reshape(E, 1, D), W1, W2)


NCH = TPW // CCH


def _combine_body(ys_hbm, pos0_hbm, pos1_hbm, w0_hbm, w1_hbm, y_hbm,
                  buf0, buf1, outb, p0a, p1a, w0m, w1m, gsem, osem):
    wid = lax.axis_index("s") * 2 + lax.axis_index("c")
    base = wid * TPW
    pltpu.sync_copy(pos0_hbm.at[pl.ds(base, TPW)], p0a)
    pltpu.sync_copy(pos1_hbm.at[pl.ds(base, TPW)], p1a)
    pltpu.sync_copy(w0_hbm.at[pl.ds(base * E, TPW * E)], w0m)
    pltpu.sync_copy(w1_hbm.at[pl.ds(base * E, TPW * E)], w1m)

    def fire(c):
        s = c % 2
        g0 = pltpu.async_copy(ys_hbm.at[p0a.at[pl.ds(c * CCH, CCH)]],
                              buf0.at[s], gsem.at[s])
        g1 = pltpu.async_copy(ys_hbm.at[p1a.at[pl.ds(c * CCH, CCH)]],
                              buf1.at[s], gsem.at[s])
        return (g0, g1)

    pend = {0: fire(0)}
    owr = {}
    for c in range(NCH):
        s = c % 2
        if c + 1 < NCH:
            pend[c + 1] = fire(c + 1)
        g0, g1 = pend.pop(c)
        g0.wait()
        g1.wait()
        if c - 2 in owr:
            owr.pop(c - 2).wait()

        def tbody(t, carry, c=c, s=s):
            wb0 = w0m[pl.ds((c * CCH + t) * E, 16)]
            wb1 = w1m[pl.ds((c * CCH + t) * E, 16)]
            for cc in range(D // 16):
                sl = pl.ds(cc * 16, 16)
                outb[s, t, sl] = wb0 * buf0[s, t, sl] + wb1 * buf1[s, t, sl]
            return carry

        lax.fori_loop(0, CCH, tbody, 0)
        owr[c] = pltpu.async_copy(
            outb.at[s], y_hbm.at[pl.ds(base + c * CCH, CCH)], osem.at[s])
    for c in sorted(owr):
        owr.pop(c).wait()


def _combine(ys, pos0, pos1, w0, w1):
    mesh = plsc.VectorSubcoreMesh(core_axis_name="c", subcore_axis_name="s")
    f = functools.partial(
        pl.kernel, _combine_body, mesh=mesh,
        out_type=jax.ShapeDtypeStruct((N, D), jnp.float32),
        scratch_types=[
            pltpu.VMEM((2, CCH, D), jnp.float32),
            pltpu.VMEM((2, CCH, D), jnp.float32),
            pltpu.VMEM((2, CCH, D), jnp.float32),
            pltpu.VMEM((TPW,), jnp.int32),
            pltpu.VMEM((TPW,), jnp.int32),
            pltpu.VMEM((TPW * E,), jnp.float32),
            pltpu.VMEM((TPW * E,), jnp.float32),
            pltpu.SemaphoreType.DMA((2,)),
            pltpu.SemaphoreType.DMA((2,)),
        ],
        compiler_params=pltpu.CompilerParams(needs_layout_passes=False),
    )()
    return f(ys, pos0, pos1, w0, w1)


def kernel(x, Wg, bg, W1, b1, W2, b2):
    (probs, mask, i0, i1, r0, r1, w0, w1, counts, aux) = _gating(x, Wg, bg)

    counts = counts[0]
    padded = ((counts + B - 1) // B) * B
    csum = jnp.cumsum(padded)
    off = (csum - padded).astype(jnp.int32)
    blk_end = (csum // B).astype(jnp.int32)
    bids = jnp.arange(NB, dtype=jnp.int32)
    block_expert = jnp.minimum(
        jnp.sum((bids[:, None] >= blk_end[None, :]).astype(jnp.int32), axis=1),
        E - 1).astype(jnp.int32)
    used = blk_end[-1]
    block_expert = jnp.where(bids < used, block_expert, block_expert[used - 1])

    i0f = i0.reshape(N)
    i1f = i1.reshape(N)
    r0f = r0.reshape(N)
    r1f = r1.reshape(N)

    xs, pos0, pos1 = _dispatch(x, i0f, i1f, r0f, r1f, off)
    ys = _mlp(block_expert, used, xs, W1, b1, W2, b2)
    y = _combine(ys, pos0, pos1, w0.reshape(N * E), w1.reshape(N * E))

    return (y, aux.reshape(()), probs, mask)


# speedup vs baseline: 2.6275x; 1.0205x over previous
import functools

import jax
import jax.numpy as jnp
from jax import lax
from jax.experimental import pallas as pl
from jax.experimental.pallas import tpu as pltpu
from jax.experimental.pallas import tpu_sc as plsc

N = 4096
D = 1024
E = 16
K = 2
H = 2048

TB = 256
NG = N // TB
B = 128
NB = N * K // B + E
PMAX = NB * B

NWORK = 32
TPW = N // NWORK
DCH = 32
CCH = 16


def _gating_body(x_ref, wg_ref, bg_ref,
                 probs_ref, mask_ref, i0_ref, i1_ref, r0_ref, r1_ref,
                 w0_ref, w1_ref, counts_ref, aux_ref,
                 run_ref, acc_ref):
    b = pl.program_id(0)

    @pl.when(b == 0)
    def _init():
        run_ref[...] = jnp.zeros_like(run_ref)
        acc_ref[...] = jnp.zeros_like(acc_ref)

    x = x_ref[...]
    logits = jnp.dot(x, wg_ref[...], preferred_element_type=jnp.float32)
    logits = logits + bg_ref[...]
    m = jnp.max(logits, axis=1, keepdims=True)
    p = jnp.exp(logits - m)
    probs = p / jnp.sum(p, axis=1, keepdims=True)
    probs_ref[...] = probs

    iota = lax.broadcasted_iota(jnp.int32, (TB, E), 1)
    m0 = jnp.max(probs, axis=1, keepdims=True)
    i0 = jnp.min(jnp.where(probs == m0, iota, E), axis=1, keepdims=True)
    oh0 = iota == i0
    probs_m = jnp.where(oh0, -1.0, probs)
    m1 = jnp.max(probs_m, axis=1, keepdims=True)
    i1 = jnp.min(jnp.where(probs_m == m1, iota, E), axis=1, keepdims=True)
    oh1 = iota == i1

    mask_ref[...] = (oh0 | oh1).astype(jnp.float32)
    s = m0 + m1 + 1e-9
    w0_ref[...] = (m0 / s) + jnp.zeros((TB, E), jnp.float32)
    w1_ref[...] = (m1 / s) + jnp.zeros((TB, E), jnp.float32)
    i0_ref[...] = i0
    i1_ref[...] = i1

    oh0f = oh0.astype(jnp.float32)
    oh1f = oh1.astype(jnp.float32)
    sel = oh0f + oh1f
    ri = lax.broadcasted_iota(jnp.int32, (TB, TB), 0)
    ci = lax.broadcasted_iota(jnp.int32, (TB, TB), 1)
    tril = (ci < ri).astype(jnp.float32)
    excl = jnp.dot(tril, sel, preferred_element_type=jnp.float32) + run_ref[...]
    r0_ref[...] = jnp.sum(excl * oh0f, axis=1, keepdims=True).astype(jnp.int32)
    r1_ref[...] = jnp.sum(excl * oh1f, axis=1, keepdims=True).astype(jnp.int32)
    run_ref[...] = run_ref[...] + jnp.sum(sel, axis=0, keepdims=True)
    acc_ref[...] = acc_ref[...] + jnp.sum(probs, axis=0, keepdims=True)
    counts_ref[...] = run_ref[...].astype(jnp.int32)

    @pl.when(b == NG - 1)
    def _fin():
        mean = acc_ref[...] / N
        u = 1.0 / E
        aux = jnp.sum(u * (jnp.log(u) - jnp.log(mean))) / E
        aux_ref[...] = jnp.full((1, 1), aux, jnp.float32)


def _gating(x, Wg, bg):
    out_shapes = (
        jax.ShapeDtypeStruct((N, E), jnp.float32),
        jax.ShapeDtypeStruct((N, E), jnp.float32),
        jax.ShapeDtypeStruct((N, 1), jnp.int32),
        jax.ShapeDtypeStruct((N, 1), jnp.int32),
        jax.ShapeDtypeStruct((N, 1), jnp.int32),
        jax.ShapeDtypeStruct((N, 1), jnp.int32),
        jax.ShapeDtypeStruct((N, E), jnp.float32),
        jax.ShapeDtypeStruct((N, E), jnp.float32),
        jax.ShapeDtypeStruct((1, E), jnp.int32),
        jax.ShapeDtypeStruct((1, 1), jnp.float32),
    )
    ne_spec = pl.BlockSpec((TB, E), lambda b: (b, 0))
    n1_spec = pl.BlockSpec((TB, 1), lambda b: (b, 0))
    one_spec = pl.BlockSpec((1, E), lambda b: (0, 0))
    return pl.pallas_call(
        _gating_body,
        grid=(NG,),
        in_specs=[
            pl.BlockSpec((TB, D), lambda b: (b, 0)),
            pl.BlockSpec((D, E), lambda b: (0, 0)),
            one_spec,
        ],
        out_specs=(ne_spec, ne_spec, n1_spec, n1_spec, n1_spec, n1_spec,
                   ne_spec, ne_spec, one_spec,
                   pl.BlockSpec((1, 1), lambda b: (0, 0))),
        out_shape=out_shapes,
        scratch_shapes=[
            pltpu.VMEM((1, E), jnp.float32),
            pltpu.VMEM((1, E), jnp.float32),
        ],
    )(x, Wg, bg.reshape(1, E))


NDC = TPW // DCH


def _dispatch_body(x_hbm, i0_hbm, i1_hbm, r0_hbm, r1_hbm, off_hbm,
                   xs_hbm, pos0_hbm, pos1_hbm,
                   xrows, e0a, e1a, r0a, r1a, p0b, p1b, off_v, sem, psem):
    wid = lax.axis_index("s") * 2 + lax.axis_index("c")
    base = wid * TPW
    pltpu.sync_copy(off_hbm, off_v)
    pltpu.sync_copy(i0_hbm.at[pl.ds(base, TPW)], e0a)
    pltpu.sync_copy(i1_hbm.at[pl.ds(base, TPW)], e1a)
    pltpu.sync_copy(r0_hbm.at[pl.ds(base, TPW)], r0a)
    pltpu.sync_copy(r1_hbm.at[pl.ds(base, TPW)], r1a)
    for c in range(NDC):
        for i in range(DCH // 16):
            sl = pl.ds(c * DCH + i * 16, 16)
            dsl = pl.ds(i * 16, 16)
            p0b[c, dsl] = plsc.load_gather(off_v, [e0a[sl]]) + r0a[sl]
            p1b[c, dsl] = plsc.load_gather(off_v, [e1a[sl]]) + r1a[sl]

    pwr = []
    for c in range(NDC):
        pwr.append(pltpu.async_copy(
            p0b.at[c], pos0_hbm.at[pl.ds(base + c * DCH, DCH)], psem))
        pwr.append(pltpu.async_copy(
            p1b.at[c], pos1_hbm.at[pl.ds(base + c * DCH, DCH)], psem))

    pend = {}
    for c in range(NDC):
        s = c % 2
        if c - 2 in pend:
            d0, d1 = pend.pop(c - 2)
            d0.wait()
            d1.wait()
        pltpu.sync_copy(x_hbm.at[pl.ds(base + c * DCH, DCH)], xrows.at[s])
        pend[c] = (pltpu.async_copy(xrows.at[s], xs_hbm.at[p0b.at[c]], sem.at[s]),
                   pltpu.async_copy(xrows.at[s], xs_hbm.at[p1b.at[c]], sem.at[s]))
    for c in sorted(pend):
        d0, d1 = pend.pop(c)
        d0.wait()
        d1.wait()
    for w in pwr:
        w.wait()


def _dispatch(x, i0, i1, r0, r1, off):
    mesh = plsc.VectorSubcoreMesh(core_axis_name="c", subcore_axis_name="s")
    f = functools.partial(
        pl.kernel, _dispatch_body, mesh=mesh,
        out_type=(jax.ShapeDtypeStruct((PMAX, D), jnp.float32),
                  jax.ShapeDtypeStruct((N,), jnp.int32),
                  jax.ShapeDtypeStruct((N,), jnp.int32)),
        scratch_types=[
            pltpu.VMEM((2, DCH, D), jnp.float32),
            pltpu.VMEM((TPW,), jnp.int32),
            pltpu.VMEM((TPW,), jnp.int32),
            pltpu.VMEM((TPW,), jnp.int32),
            pltpu.VMEM((TPW,), jnp.int32),
            pltpu.VMEM((NDC, DCH), jnp.int32),
            pltpu.VMEM((NDC, DCH), jnp.int32),
            pltpu.VMEM((E,), jnp.int32),
            pltpu.SemaphoreType.DMA((2,)),
            pltpu.SemaphoreType.DMA,
        ],
        compiler_params=pltpu.CompilerParams(needs_layout_passes=False),
    )()
    return f(x, i0, i1, r0, r1, off)


NSLOT = 3
EV = 32


def _mlp_body(be_r, tr_r, sl_r, pfe_r, pfv_r, pfs_r, ev_r,
              xs_ref, b1_ref, b2_ref, w1_any, w2_any, ys_ref,
              w1b, w2b, sems):
    b = pl.program_id(0)
    nv = ev_r[EV]
    used = ev_r[EV + 1]

    def fetch(e, s):
        pltpu.make_async_copy(w1_any.at[e], w1b.at[s], sems.at[s]).start()
        pltpu.make_async_copy(w2_any.at[e], w2b.at[s], sems.at[s]).start()

    @pl.when(b == 0)
    def _prime():
        fetch(ev_r[0], 0)

        @pl.when(nv >= 2)
        def _p1():
            fetch(ev_r[1], 1)

        @pl.when(nv >= 3)
        def _p2():
            fetch(ev_r[2], 2)

    sl = sl_r[b]

    @pl.when(tr_r[b] == 1)
    def _on_transition():
        @pl.when((b > 0) & (pfv_r[b] == 1))
        def _pf():
            fetch(pfe_r[b], pfs_r[b])

        pltpu.make_async_copy(w1_any.at[be_r[b]], w1b.at[sl], sems.at[sl]).wait()
        pltpu.make_async_copy(w2_any.at[be_r[b]], w2b.at[sl], sems.at[sl]).wait()

    @pl.when(b < used)
    def _compute():
        x = xs_ref[...]
        h = jnp.dot(x, w1b[sl], preferred_element_type=jnp.float32)
        h = jnp.maximum(h + b1_ref[0], 0.0)
        ys_ref[...] = (jnp.dot(h, w2b[sl], preferred_element_type=jnp.float32)
                       + b2_ref[0])


def _mlp(block_expert, used, xs, W1, b1, W2, b2):
    i32 = jnp.int32
    trans = jnp.concatenate([
        jnp.ones((1,), i32),
        (block_expert[1:] != block_expert[:-1]).astype(i32)])
    visit = jnp.cumsum(trans) - 1
    slot = (visit % NSLOT).astype(i32)
    nv = visit[-1] + 1
    ev = jnp.zeros((EV,), i32).at[jnp.minimum(visit, EV - 1)].set(block_expert)
    pf_v = visit + 2
    pf_valid = (pf_v < nv).astype(i32)
    pf_e = ev[jnp.minimum(pf_v, EV - 1)]
    pf_slot = (pf_v % NSLOT).astype(i32)
    evnv = jnp.concatenate([ev, nv.reshape(1), used.reshape(1)]).astype(i32)

    def data_idx(b, *s):
        ev_r = s[6]
        return (jnp.minimum(b, ev_r[EV + 1] - 1), 0)

    grid_spec = pltpu.PrefetchScalarGridSpec(
        num_scalar_prefetch=7,
        grid=(NB,),
        in_specs=[
            pl.BlockSpec((B, D), data_idx),
            pl.BlockSpec((1, 1, H), lambda b, be, *s: (be[b], 0, 0)),
            pl.BlockSpec((1, 1, D), lambda b, be, *s: (be[b], 0, 0)),
            pl.BlockSpec(memory_space=pl.ANY),
            pl.BlockSpec(memory_space=pl.ANY),
        ],
        out_specs=pl.BlockSpec((B, D), data_idx),
        scratch_shapes=[
            pltpu.VMEM((NSLOT, D, H), jnp.float32),
            pltpu.VMEM((NSLOT, H, D), jnp.float32),
            pltpu.SemaphoreType.DMA((NSLOT,)),
        ],
    )
    return pl.pallas_call(
        _mlp_body,
        grid_spec=grid_spec,
        out_shape=jax.ShapeDtypeStruct((PMAX, D), jnp.float32),
        compiler_params=pltpu.CompilerParams(vmem_limit_bytes=110 * 1024 * 1024),
    )(block_expert, trans, slot, pf_e, pf_valid, pf_slot, evnv,
      xs, b1.reshape(E, 1, H), b2.reshape(E, 1, D), W1, W2)


NCH = TPW // CCH


def _combine_body(ys_hbm, pos0_hbm, pos1_hbm, w0_hbm, w1_hbm, y_hbm,
                  buf0, buf1, outb, p0a, p1a, w0m, w1m, gsem, osem):
    wid = lax.axis_index("s") * 2 + lax.axis_index("c")
    base = wid * TPW
    pltpu.sync_copy(pos0_hbm.at[pl.ds(base, TPW)], p0a)
    pltpu.sync_copy(pos1_hbm.at[pl.ds(base, TPW)], p1a)
    pltpu.sync_copy(w0_hbm.at[pl.ds(base * E, TPW * E)], w0m)
    pltpu.sync_copy(w1_hbm.at[pl.ds(base * E, TPW * E)], w1m)

    def fire(c):
        s = c % 2
        g0 = pltpu.async_copy(ys_hbm.at[p0a.at[pl.ds(c * CCH, CCH)]],
                              buf0.at[s], gsem.at[s])
        g1 = pltpu.async_copy(ys_hbm.at[p1a.at[pl.ds(c * CCH, CCH)]],
                              buf1.at[s], gsem.at[s])
        return (g0, g1)

    pend = {0: fire(0)}
    owr = {}
    for c in range(NCH):
        s = c % 2
        if c + 1 < NCH:
            pend[c + 1] = fire(c + 1)
        g0, g1 = pend.pop(c)
        g0.wait()
        g1.wait()
        if c - 2 in owr:
            owr.pop(c - 2).wait()

        def tbody(t, carry, c=c, s=s):
            wb0 = w0m[pl.ds((c * CCH + t) * E, 16)]
            wb1 = w1m[pl.ds((c * CCH + t) * E, 16)]
            for cc in range(D // 16):
                sl = pl.ds(cc * 16, 16)
                outb[s, t, sl] = wb0 * buf0[s, t, sl] + wb1 * buf1[s, t, sl]
            return carry

        lax.fori_loop(0, CCH, tbody, 0)
        owr[c] = pltpu.async_copy(
            outb.at[s], y_hbm.at[pl.ds(base + c * CCH, CCH)], osem.at[s])
    for c in sorted(owr):
        owr.pop(c).wait()


def _combine(ys, pos0, pos1, w0, w1):
    mesh = plsc.VectorSubcoreMesh(core_axis_name="c", subcore_axis_name="s")
    f = functools.partial(
        pl.kernel, _combine_body, mesh=mesh,
        out_type=jax.ShapeDtypeStruct((N, D), jnp.float32),
        scratch_types=[
            pltpu.VMEM((2, CCH, D), jnp.float32),
            pltpu.VMEM((2, CCH, D), jnp.float32),
            pltpu.VMEM((2, CCH, D), jnp.float32),
            pltpu.VMEM((TPW,), jnp.int32),
            pltpu.VMEM((TPW,), jnp.int32),
            pltpu.VMEM((TPW * E,), jnp.float32),
            pltpu.VMEM((TPW * E,), jnp.float32),
            pltpu.SemaphoreType.DMA((2,)),
            pltpu.SemaphoreType.DMA((2,)),
        ],
        compiler_params=pltpu.CompilerParams(needs_layout_passes=False),
    )()
    return f(ys, pos0, pos1, w0, w1)


def kernel(x, Wg, bg, W1, b1, W2, b2):
    (probs, mask, i0, i1, r0, r1, w0, w1, counts, aux) = _gating(x, Wg, bg)

    counts = counts[0]
    padded = ((counts + B - 1) // B) * B
    csum = jnp.cumsum(padded)
    off = (csum - padded).astype(jnp.int32)
    blk_end = (csum // B).astype(jnp.int32)
    bids = jnp.arange(NB, dtype=jnp.int32)
    block_expert = jnp.minimum(
        jnp.sum((bids[:, None] >= blk_end[None, :]).astype(jnp.int32), axis=1),
        E - 1).astype(jnp.int32)
    used = blk_end[-1]
    block_expert = jnp.where(bids < used, block_expert, block_expert[used - 1])

    i0f = i0.reshape(N)
    i1f = i1.reshape(N)
    r0f = r0.reshape(N)
    r1f = r1.reshape(N)

    xs, pos0, pos1 = _dispatch(x, i0f, i1f, r0f, r1f, off)
    ys = _mlp(block_expert, used, xs, W1, b1, W2, b2)
    y = _combine(ys, pos0, pos1, w0.reshape(N * E), w1.reshape(N * E))

    return (y, aux.reshape(()), probs, mask)
